# Initial kernel scaffold; baseline (speedup 1.0000x reference)
import jax
import jax.numpy as jnp
from jax.experimental import pallas as pl


def kernel(input, graph_indices, node_counts, W1, W2):
    raise NotImplementedError("write your pallas kernel here")


# speedup vs baseline: 2.1664x; 2.1664x over previous
import functools

import jax
import jax.numpy as jnp
from jax import lax
from jax.experimental import pallas as pl
from jax.experimental.pallas import tpu as pltpu
from jax.experimental.pallas import tpu_sc as plsc

N = 100000
D = 512
G = 1000


_BLK = 1000

def _dense_body(x_ref, w1_ref, w2_ref, o_ref):
    x = x_ref[...]
    a = jnp.dot(x, w1_ref[...], preferred_element_type=jnp.float32)
    b = jnp.dot(x, w2_ref[...], preferred_element_type=jnp.float32)
    z = jnp.where(a > 0, a, jnp.exp(a) - 1.0)
    o_ref[...] = z * b


def _dense(x, w1t, w2t):
    grid = (N // _BLK,)
    return pl.pallas_call(
        _dense_body,
        grid=grid,
        in_specs=[
            pl.BlockSpec((_BLK, D), lambda i: (i, 0)),
            pl.BlockSpec((D, D), lambda i: (0, 0)),
            pl.BlockSpec((D, D), lambda i: (0, 0)),
        ],
        out_specs=pl.BlockSpec((_BLK, D), lambda i: (i, 0)),
        out_shape=jax.ShapeDtypeStruct((N, D), jnp.float32),
    )(x, w1t, w2t)


_NC = 2
_NS = 16
_K = 80
_NCHUNKS = N // _K
_GP = 1024
_CS = 4
_RG = _NC * _NS // _CS
_DS = D // _CS
_ZR = _CS * _GP // _NS


def _seg_body(gated_hbm, idx_hbm, zeros_hbm, out_hbm, idx_v, rows_v, acc_sh):
    c = lax.axis_index("c")
    s = lax.axis_index("s")
    ci = s % _CS
    ri = c * (_NS // _CS) + s // _CS

    pltpu.sync_copy(zeros_hbm, acc_sh.at[pl.ds(s * _ZR, _ZR)])
    plsc.subcore_barrier()

    nc = (_NCHUNKS // _RG) + jnp.where(ri < _NCHUNKS % _RG, 1, 0)

    def body(j, carry):
        base = (ri + j * _RG) * _K
        pltpu.sync_copy(idx_hbm.at[pl.ds(base, _K)], idx_v)
        pltpu.sync_copy(gated_hbm.at[pl.ds(base, _K), pl.ds(ci * _DS, _DS)],
                        rows_v)
        for i in range(_K // 16):
            idx_v[pl.ds(i * 16, 16)] = idx_v[pl.ds(i * 16, 16)] + ci * _GP
        pltpu.sync_copy(rows_v, acc_sh.at[idx_v], add=True)
        return carry

    lax.fori_loop(0, nc, body, 0)
    plsc.subcore_barrier()

    pltpu.sync_copy(acc_sh.at[pl.ds(s * _ZR, _ZR)],
                    out_hbm.at[c, pl.ds(s * _ZR, _ZR)])


@functools.cache
def _make_seg_kernel():
    return pl.kernel(
        _seg_body,
        out_type=jax.ShapeDtypeStruct((_NC, _CS * _GP, _DS), jnp.float32),
        mesh=plsc.VectorSubcoreMesh(core_axis_name="c", subcore_axis_name="s",
                                    num_cores=_NC, num_subcores=_NS),
        scratch_types=[
            pltpu.VMEM((_K,), jnp.int32),
            pltpu.VMEM((_K, _DS), jnp.float32),
            pltpu.VMEM_SHARED((_CS * _GP, _DS), jnp.float32),
        ],
    )


def kernel(input, graph_indices, node_counts, W1, W2):
    del node_counts
    gated = _dense(input, W1.T, W2.T)
    zeros = jnp.zeros((_ZR, _DS), jnp.float32)
    partials = _make_seg_kernel()(gated, graph_indices.astype(jnp.int32), zeros)
    acc = partials[0] + partials[1]
    acc = acc.reshape(_CS, _GP, _DS).transpose(1, 0, 2)
    return acc.reshape(_GP, D)[:G]


# speedup vs baseline: 3.1999x; 1.4771x over previous
import functools

import jax
import jax.numpy as jnp
from jax import lax
from jax.experimental import pallas as pl
from jax.experimental.pallas import tpu as pltpu
from jax.experimental.pallas import tpu_sc as plsc

N = 100000
D = 512
G = 1000


_BLK = 1000

def _dense_body(x_ref, w1_ref, w2_ref, o_ref):
    x = x_ref[...]
    a = jnp.dot(x, w1_ref[...], preferred_element_type=jnp.float32)
    b = jnp.dot(x, w2_ref[...], preferred_element_type=jnp.float32)
    z = jnp.where(a > 0, a, jnp.exp(a) - 1.0)
    o_ref[...] = z * b


def _dense(x, w1t, w2t):
    grid = (N // _BLK,)
    return pl.pallas_call(
        _dense_body,
        grid=grid,
        in_specs=[
            pl.BlockSpec((_BLK, D), lambda i: (i, 0)),
            pl.BlockSpec((D, D), lambda i: (0, 0)),
            pl.BlockSpec((D, D), lambda i: (0, 0)),
        ],
        out_specs=pl.BlockSpec((_BLK, D), lambda i: (i, 0)),
        out_shape=jax.ShapeDtypeStruct((N, D), jnp.float32),
    )(x, w1t, w2t)


_NC = 2
_NS = 16
_K = 80
_NCHUNKS = N // _K
_GP = 1024
_CS = 4
_RG = _NC * _NS // _CS
_DS = D // _CS
_ZR = _CS * _GP // _NS


_NMAIN = (_NCHUNKS // _RG // 2) * 2 * _RG
_JMAIN = _NMAIN // _RG


def _seg_body(gated_hbm, idx_hbm, zeros_hbm, out_hbm,
              idx_v, rows_v, acc_sh, sem_i, sem_r):
    c = lax.axis_index("c")
    s = lax.axis_index("s")
    ci = s % _CS
    ri = c * (_NS // _CS) + s // _CS

    pltpu.sync_copy(zeros_hbm, acc_sh.at[pl.ds(s * _ZR, _ZR)])
    plsc.subcore_barrier()

    def start_loads(j, b):
        base = (ri + j * _RG) * _K
        pltpu.async_copy(idx_hbm.at[pl.ds(base, _K)], idx_v.at[b], sem_i.at[b])
        pltpu.async_copy(gated_hbm.at[pl.ds(base, _K), pl.ds(ci * _DS, _DS)],
                         rows_v.at[b], sem_r.at[b])

    def wait_loads(b):
        pltpu.make_async_copy(idx_hbm.at[pl.ds(0, _K)], idx_v.at[b],
                              sem_i.at[b]).wait()
        pltpu.make_async_copy(gated_hbm.at[pl.ds(0, _K), pl.ds(0, _DS)],
                              rows_v.at[b], sem_r.at[b]).wait()

    def scatter(b):
        for i in range(_K // 16):
            idx_v[b, pl.ds(i * 16, 16)] = idx_v[b, pl.ds(i * 16, 16)] + ci * _GP
        pltpu.sync_copy(rows_v.at[b], acc_sh.at[idx_v.at[b]], add=True)

    start_loads(0, 0)

    @pl.loop(0, _JMAIN, step=2)
    def _pipe(jj):
        for b in range(2):
            j = jj + b

            @pl.when(j + 1 < _JMAIN)
            def _():
                start_loads(j + 1, 1 - b)

            wait_loads(b)
            scatter(b)

    @pl.when(ri < _NCHUNKS - _NMAIN)
    def _():
        base = (_NMAIN + ri) * _K
        pltpu.sync_copy(idx_hbm.at[pl.ds(base, _K)], idx_v.at[0])
        pltpu.sync_copy(gated_hbm.at[pl.ds(base, _K), pl.ds(ci * _DS, _DS)],
                        rows_v.at[0])
        scatter(0)

    plsc.subcore_barrier()

    pltpu.sync_copy(
        acc_sh.at[pl.ds(s * _ZR, _ZR)],
        out_hbm.at[c, pl.ds((s % 4) * _ZR, _ZR), pl.ds((s // 4) * _DS, _DS)])


@functools.cache
def _make_seg_kernel():
    return pl.kernel(
        _seg_body,
        out_type=jax.ShapeDtypeStruct((_NC, _GP, D), jnp.float32),
        mesh=plsc.VectorSubcoreMesh(core_axis_name="c", subcore_axis_name="s",
                                    num_cores=_NC, num_subcores=_NS),
        scratch_types=[
            pltpu.VMEM((2, _K), jnp.int32),
            pltpu.VMEM((2, _K, _DS), jnp.float32),
            pltpu.VMEM_SHARED((_CS * _GP, _DS), jnp.float32),
            pltpu.SemaphoreType.DMA((2,)),
            pltpu.SemaphoreType.DMA((2,)),
        ],
    )


def kernel(input, graph_indices, node_counts, W1, W2):
    del node_counts
    gated = _dense(input, W1.T, W2.T)
    zeros = jnp.zeros((_ZR, _DS), jnp.float32)
    partials = _make_seg_kernel()(gated, graph_indices.astype(jnp.int32), zeros)
    return (partials[0] + partials[1])[:G]
